# baseline (speedup 1.0000x reference)
import functools

import jax
import jax.numpy as jnp
from jax import lax
from jax.experimental import pallas as pl
from jax.experimental.pallas import tpu as pltpu
from jax.experimental.pallas import tpu_sc as plsc

N = 10000
NPAD = 10240
DIN = 128
DH = 128
DOUT = 64
E = 320000
NCORES = 2
NSUB = 16
NW = NCORES * NSUB
LANES = 128
CHUNK = 128
NCHUNK = 80
NSEG = 2
SEG = NCHUNK // NSEG
EPT = NCHUNK * CHUNK
EPAD = NW * EPT
RPT = NPAD // NSUB

_sc_mesh = plsc.VectorSubcoreMesh(core_axis_name="c", subcore_axis_name="s")


@functools.partial(
    pl.kernel,
    mesh=_sc_mesh,
    out_type=jax.ShapeDtypeStruct((NCORES, NPAD, LANES), jnp.float32),
    scratch_types=[
        pltpu.VMEM((NCHUNK, CHUNK), jnp.int32),
        pltpu.VMEM((CHUNK, LANES), jnp.float32),
        pltpu.VMEM_SHARED((NPAD, LANES), jnp.float32),
    ],
)
def _sc_count(dst_hbm, out_hbm, dst_v, ones_v, cnt_sh):
    c = lax.axis_index("c")
    s = lax.axis_index("s")
    wid = s * NCORES + c

    z16 = jnp.zeros((16,), jnp.float32)

    def _zero(r, _):
        for j in range(LANES // 16):
            ones_v[r, pl.ds(j * 16, 16)] = z16
        return 0

    lax.fori_loop(0, CHUNK, _zero, 0)
    for k in range(RPT // CHUNK):
        pltpu.sync_copy(ones_v, cnt_sh.at[pl.ds(s * RPT + k * CHUNK, CHUNK)])

    o16 = jnp.ones((16,), jnp.float32)

    def _fill(r, _):
        for j in range(LANES // 16):
            ones_v[r, pl.ds(j * 16, 16)] = o16
        return 0

    lax.fori_loop(0, CHUNK, _fill, 0)
    pltpu.sync_copy(dst_hbm.at[wid], dst_v)
    plsc.subcore_barrier()

    def _body(i, _):
        pltpu.sync_copy(ones_v, cnt_sh.at[dst_v.at[i]], add=True)
        return 0

    lax.fori_loop(0, NCHUNK, _body, 0)
    plsc.subcore_barrier()
    pltpu.sync_copy(cnt_sh.at[pl.ds(s * RPT, RPT)],
                    out_hbm.at[c, pl.ds(s * RPT, RPT)])


def _make_agg(D):
    @functools.partial(
        pl.kernel,
        mesh=_sc_mesh,
        out_type=jax.ShapeDtypeStruct((NCORES, NPAD, D), jnp.float32),
        scratch_types=[
            pltpu.VMEM((SEG, CHUNK), jnp.int32),
            pltpu.VMEM((SEG, CHUNK), jnp.int32),
            pltpu.VMEM((CHUNK, D), jnp.float32),
            pltpu.VMEM((CHUNK, D), jnp.float32),
            pltpu.VMEM_SHARED((NPAD, D), jnp.float32),
            pltpu.SemaphoreType.DMA,
            pltpu.SemaphoreType.DMA,
            pltpu.SemaphoreType.DMA,
            pltpu.SemaphoreType.DMA,
        ],
    )
    def _agg(g_hbm, src_hbm, dst_hbm, out_hbm, src_v, dst_v, rows0_v, rows1_v,
             acc_sh, sem0, sem1, ssem0, ssem1):
        c = lax.axis_index("c")
        s = lax.axis_index("s")
        wid = s * NCORES + c

        z16 = jnp.zeros((16,), jnp.float32)

        def _zero(r, _):
            for j in range(D // 16):
                rows0_v[r, pl.ds(j * 16, 16)] = z16
            return 0

        lax.fori_loop(0, CHUNK, _zero, 0)
        for k in range(RPT // CHUNK):
            pltpu.sync_copy(rows0_v, acc_sh.at[pl.ds(s * RPT + k * CHUNK, CHUNK)])
        plsc.subcore_barrier()

        for seg in range(NSEG):
            pltpu.sync_copy(src_hbm.at[wid, pl.ds(seg * SEG, SEG)], src_v)
            pltpu.sync_copy(dst_hbm.at[wid, pl.ds(seg * SEG, SEG)], dst_v)

            def _body(i, _):
                pltpu.async_copy(g_hbm.at[src_v.at[i]], rows0_v, sem0).wait()
                pltpu.sync_copy(rows0_v, acc_sh.at[dst_v.at[i]], add=True)
                return 0

            lax.fori_loop(0, SEG, _body, 0)
        plsc.subcore_barrier()
        pltpu.sync_copy(acc_sh.at[pl.ds(s * RPT, RPT)],
                        out_hbm.at[c, pl.ds(s * RPT, RPT)])

    return _agg


_agg_h = _make_agg(DH)


BLK = 1024
GRID = NPAD // BLK


def _tc_mm_body(x_ref, w1_ref, h_ref):
    h_ref[...] = jnp.dot(x_ref[...], w1_ref[...],
                         preferred_element_type=jnp.float32)


_tc_mm = pl.pallas_call(
    _tc_mm_body,
    grid=(GRID,),
    in_specs=[
        pl.BlockSpec((BLK, DIN), lambda i: (i, 0)),
        pl.BlockSpec((DIN, DH), lambda i: (0, 0)),
    ],
    out_specs=pl.BlockSpec((BLK, DH), lambda i: (i, 0)),
    out_shape=jax.ShapeDtypeStruct((NPAD, DH), jnp.float32),
)


def _tc_a_body(h_ref, c0_ref, c1_ref, g_ref, dinv_ref):
    deg = 1.0 + c0_ref[:, 0:1] + c1_ref[:, 0:1]
    dinv = lax.rsqrt(deg)
    g_ref[...] = h_ref[...] * dinv
    dinv_ref[...] = dinv


_tc_a = pl.pallas_call(
    _tc_a_body,
    grid=(GRID,),
    in_specs=[
        pl.BlockSpec((BLK, DH), lambda i: (i, 0)),
        pl.BlockSpec((BLK, LANES), lambda i: (i, 0)),
        pl.BlockSpec((BLK, LANES), lambda i: (i, 0)),
    ],
    out_specs=[
        pl.BlockSpec((BLK, DH), lambda i: (i, 0)),
        pl.BlockSpec((BLK, 1), lambda i: (i, 0)),
    ],
    out_shape=[
        jax.ShapeDtypeStruct((NPAD, DH), jnp.float32),
        jax.ShapeDtypeStruct((NPAD, 1), jnp.float32),
    ],
)


def _tc_b_body(p0_ref, p1_ref, g1_ref, dinv_ref, b1_ref, w2_ref, g2_ref):
    z = (p0_ref[...] + p1_ref[...] + g1_ref[...]) * dinv_ref[...] + b1_ref[...]
    z = jnp.maximum(z, 0.0)
    h2 = jnp.dot(z, w2_ref[...], preferred_element_type=jnp.float32)
    g2_ref[...] = h2 * dinv_ref[...]


_tc_b = pl.pallas_call(
    _tc_b_body,
    grid=(GRID,),
    in_specs=[
        pl.BlockSpec((BLK, DH), lambda i: (i, 0)),
        pl.BlockSpec((BLK, DH), lambda i: (i, 0)),
        pl.BlockSpec((BLK, DH), lambda i: (i, 0)),
        pl.BlockSpec((BLK, 1), lambda i: (i, 0)),
        pl.BlockSpec((1, DH), lambda i: (0, 0)),
        pl.BlockSpec((DH, DH), lambda i: (0, 0)),
    ],
    out_specs=pl.BlockSpec((BLK, DH), lambda i: (i, 0)),
    out_shape=jax.ShapeDtypeStruct((NPAD, DH), jnp.float32),
)


def _tc_c_body(q0_ref, q1_ref, g2_ref, dinv_ref, b2_ref, out_ref):
    zf = q0_ref[...] + q1_ref[...] + g2_ref[...]
    z = zf[:, :DOUT] * dinv_ref[...] + b2_ref[...]
    m = jnp.max(z, axis=1, keepdims=True)
    zs = z - m
    lse = jnp.log(jnp.sum(jnp.exp(zs), axis=1, keepdims=True))
    out_ref[...] = zs - lse


_tc_c = pl.pallas_call(
    _tc_c_body,
    grid=(GRID,),
    in_specs=[
        pl.BlockSpec((BLK, DH), lambda i: (i, 0)),
        pl.BlockSpec((BLK, DH), lambda i: (i, 0)),
        pl.BlockSpec((BLK, DH), lambda i: (i, 0)),
        pl.BlockSpec((BLK, 1), lambda i: (i, 0)),
        pl.BlockSpec((1, DOUT), lambda i: (0, 0)),
    ],
    out_specs=pl.BlockSpec((BLK, DOUT), lambda i: (i, 0)),
    out_shape=jax.ShapeDtypeStruct((NPAD, DOUT), jnp.float32),
)


def kernel(x, edge_index, W1, b1, W2, b2):
    xp = jnp.zeros((NPAD, DIN), jnp.float32).at[:N].set(x)
    src = edge_index[0].astype(jnp.int32)
    dst = edge_index[1].astype(jnp.int32)
    pad = jnp.full((EPAD - E,), N, jnp.int32)
    src3 = jnp.concatenate([src, pad]).reshape(NW, NCHUNK, CHUNK)
    dst3 = jnp.concatenate([dst, pad]).reshape(NW, NCHUNK, CHUNK)

    W2p = jnp.zeros((DH, DH), jnp.float32).at[:, :DOUT].set(W2)

    cnt = _sc_count(dst3)
    h1 = _tc_mm(xp, W1)
    g1, dinv = _tc_a(h1, cnt[0], cnt[1])
    p = _agg_h(g1, src3, dst3)
    g2 = _tc_b(p[0], p[1], g1, dinv, b1.reshape(1, DH), W2p)
    q = _agg_h(g2, src3, dst3)
    out = _tc_c(q[0], q[1], g2, dinv, b2.reshape(1, DOUT))
    return out[:N]


# speedup vs baseline: 1.0020x; 1.0020x over previous
import functools

import jax
import jax.numpy as jnp
from jax import lax
from jax.experimental import pallas as pl
from jax.experimental.pallas import tpu as pltpu
from jax.experimental.pallas import tpu_sc as plsc

N = 10000
NPAD = 10240
DIN = 128
DH = 128
DOUT = 64
E = 320000
NCORES = 2
NSUB = 16
NW = NCORES * NSUB
LANES = 128
CHUNK = 128
NCHUNK = 80
NSEG = 2
SEG = NCHUNK // NSEG
EPT = NCHUNK * CHUNK
EPAD = NW * EPT
RPT = NPAD // NSUB

_sc_mesh = plsc.VectorSubcoreMesh(core_axis_name="c", subcore_axis_name="s")


@functools.partial(
    pl.kernel,
    mesh=_sc_mesh,
    out_type=jax.ShapeDtypeStruct((NCORES, NPAD, LANES), jnp.float32),
    scratch_types=[
        pltpu.VMEM((NCHUNK, CHUNK), jnp.int32),
        pltpu.VMEM((CHUNK, LANES), jnp.float32),
        pltpu.VMEM_SHARED((NPAD, LANES), jnp.float32),
    ],
)
def _sc_count(dst_hbm, out_hbm, dst_v, ones_v, cnt_sh):
    c = lax.axis_index("c")
    s = lax.axis_index("s")
    wid = s * NCORES + c

    z16 = jnp.zeros((16,), jnp.float32)

    def _zero(r, _):
        for j in range(LANES // 16):
            ones_v[r, pl.ds(j * 16, 16)] = z16
        return 0

    lax.fori_loop(0, CHUNK, _zero, 0)
    for k in range(RPT // CHUNK):
        pltpu.sync_copy(ones_v, cnt_sh.at[pl.ds(s * RPT + k * CHUNK, CHUNK)])

    o16 = jnp.ones((16,), jnp.float32)

    def _fill(r, _):
        for j in range(LANES // 16):
            ones_v[r, pl.ds(j * 16, 16)] = o16
        return 0

    lax.fori_loop(0, CHUNK, _fill, 0)
    pltpu.sync_copy(dst_hbm.at[wid], dst_v)
    plsc.subcore_barrier()

    def _body(i, _):
        pltpu.sync_copy(ones_v, cnt_sh.at[dst_v.at[i]], add=True)
        return 0

    lax.fori_loop(0, NCHUNK, _body, 0)
    plsc.subcore_barrier()
    pltpu.sync_copy(cnt_sh.at[pl.ds(s * RPT, RPT)],
                    out_hbm.at[c, pl.ds(s * RPT, RPT)])


def _make_agg(D):
    @functools.partial(
        pl.kernel,
        mesh=_sc_mesh,
        out_type=jax.ShapeDtypeStruct((NCORES, NPAD, D), jnp.float32),
        scratch_types=[
            pltpu.VMEM((NCHUNK, CHUNK), jnp.int32),
            pltpu.VMEM((NCHUNK, CHUNK), jnp.int32),
            pltpu.VMEM((CHUNK, D), jnp.float32),
            pltpu.VMEM_SHARED((NPAD, D), jnp.float32),
            pltpu.SemaphoreType.DMA,
        ],
    )
    def _agg(g_hbm, src_hbm, dst_hbm, out_hbm, src_v, dst_v, rows0_v,
             acc_sh, sem0):
        c = lax.axis_index("c")
        s = lax.axis_index("s")
        wid = s * NCORES + c

        z16 = jnp.zeros((16,), jnp.float32)

        def _zero(r, _):
            for j in range(D // 16):
                rows0_v[r, pl.ds(j * 16, 16)] = z16
            return 0

        lax.fori_loop(0, CHUNK, _zero, 0)
        for k in range(RPT // CHUNK):
            pltpu.sync_copy(rows0_v, acc_sh.at[pl.ds(s * RPT + k * CHUNK, CHUNK)])
        plsc.subcore_barrier()

        pltpu.sync_copy(src_hbm.at[wid], src_v)
        pltpu.sync_copy(dst_hbm.at[wid], dst_v)

        def _body(i, _):
            pltpu.async_copy(g_hbm.at[src_v.at[i]], rows0_v, sem0).wait()
            pltpu.sync_copy(rows0_v, acc_sh.at[dst_v.at[i]], add=True)
            return 0

        lax.fori_loop(0, NCHUNK, _body, 0)
        plsc.subcore_barrier()
        pltpu.sync_copy(acc_sh.at[pl.ds(s * RPT, RPT)],
                        out_hbm.at[c, pl.ds(s * RPT, RPT)])

    return _agg


_agg_h = _make_agg(DH)


BLK = 1024
GRID = NPAD // BLK


def _tc_mm_body(x_ref, w1_ref, h_ref):
    h_ref[...] = jnp.dot(x_ref[...], w1_ref[...],
                         preferred_element_type=jnp.float32)


_tc_mm = pl.pallas_call(
    _tc_mm_body,
    grid=(GRID,),
    in_specs=[
        pl.BlockSpec((BLK, DIN), lambda i: (i, 0)),
        pl.BlockSpec((DIN, DH), lambda i: (0, 0)),
    ],
    out_specs=pl.BlockSpec((BLK, DH), lambda i: (i, 0)),
    out_shape=jax.ShapeDtypeStruct((NPAD, DH), jnp.float32),
)


def _tc_a_body(h_ref, c0_ref, c1_ref, g_ref, dinv_ref):
    deg = 1.0 + c0_ref[:, 0:1] + c1_ref[:, 0:1]
    dinv = lax.rsqrt(deg)
    g_ref[...] = h_ref[...] * dinv
    dinv_ref[...] = dinv


_tc_a = pl.pallas_call(
    _tc_a_body,
    grid=(GRID,),
    in_specs=[
        pl.BlockSpec((BLK, DH), lambda i: (i, 0)),
        pl.BlockSpec((BLK, LANES), lambda i: (i, 0)),
        pl.BlockSpec((BLK, LANES), lambda i: (i, 0)),
    ],
    out_specs=[
        pl.BlockSpec((BLK, DH), lambda i: (i, 0)),
        pl.BlockSpec((BLK, 1), lambda i: (i, 0)),
    ],
    out_shape=[
        jax.ShapeDtypeStruct((NPAD, DH), jnp.float32),
        jax.ShapeDtypeStruct((NPAD, 1), jnp.float32),
    ],
)


def _tc_b_body(p0_ref, p1_ref, g1_ref, dinv_ref, b1_ref, w2_ref, g2_ref):
    z = (p0_ref[...] + p1_ref[...] + g1_ref[...]) * dinv_ref[...] + b1_ref[...]
    z = jnp.maximum(z, 0.0)
    h2 = jnp.dot(z, w2_ref[...], preferred_element_type=jnp.float32)
    g2_ref[...] = h2 * dinv_ref[...]


_tc_b = pl.pallas_call(
    _tc_b_body,
    grid=(GRID,),
    in_specs=[
        pl.BlockSpec((BLK, DH), lambda i: (i, 0)),
        pl.BlockSpec((BLK, DH), lambda i: (i, 0)),
        pl.BlockSpec((BLK, DH), lambda i: (i, 0)),
        pl.BlockSpec((BLK, 1), lambda i: (i, 0)),
        pl.BlockSpec((1, DH), lambda i: (0, 0)),
        pl.BlockSpec((DH, DH), lambda i: (0, 0)),
    ],
    out_specs=pl.BlockSpec((BLK, DH), lambda i: (i, 0)),
    out_shape=jax.ShapeDtypeStruct((NPAD, DH), jnp.float32),
)


def _tc_c_body(q0_ref, q1_ref, g2_ref, dinv_ref, b2_ref, out_ref):
    zf = q0_ref[...] + q1_ref[...] + g2_ref[...]
    z = zf[:, :DOUT] * dinv_ref[...] + b2_ref[...]
    m = jnp.max(z, axis=1, keepdims=True)
    zs = z - m
    lse = jnp.log(jnp.sum(jnp.exp(zs), axis=1, keepdims=True))
    out_ref[...] = zs - lse


_tc_c = pl.pallas_call(
    _tc_c_body,
    grid=(GRID,),
    in_specs=[
        pl.BlockSpec((BLK, DH), lambda i: (i, 0)),
        pl.BlockSpec((BLK, DH), lambda i: (i, 0)),
        pl.BlockSpec((BLK, DH), lambda i: (i, 0)),
        pl.BlockSpec((BLK, 1), lambda i: (i, 0)),
        pl.BlockSpec((1, DOUT), lambda i: (0, 0)),
    ],
    out_specs=pl.BlockSpec((BLK, DOUT), lambda i: (i, 0)),
    out_shape=jax.ShapeDtypeStruct((NPAD, DOUT), jnp.float32),
)


def kernel(x, edge_index, W1, b1, W2, b2):
    xp = jnp.zeros((NPAD, DIN), jnp.float32).at[:N].set(x)
    src = edge_index[0].astype(jnp.int32)
    dst = edge_index[1].astype(jnp.int32)
    pad = jnp.full((EPAD - E,), N, jnp.int32)
    src3 = jnp.concatenate([src, pad]).reshape(NW, NCHUNK, CHUNK)
    dst3 = jnp.concatenate([dst, pad]).reshape(NW, NCHUNK, CHUNK)

    W2p = jnp.zeros((DH, DH), jnp.float32).at[:, :DOUT].set(W2)

    h1 = _tc_mm(xp, W1)
    cnt = _sc_count(dst3)
    g1, dinv = _tc_a(h1, cnt[0], cnt[1])
    p = _agg_h(g1, src3, dst3)
    g2 = _tc_b(p[0], p[1], g1, dinv, b1.reshape(1, DH), W2p)
    q = _agg_h(g2, src3, dst3)
    out = _tc_c(q[0], q[1], g2, dinv, b2.reshape(1, DOUT))
    return out[:N]


# speedup vs baseline: 1.0027x; 1.0007x over previous
import functools

import jax
import jax.numpy as jnp
from jax import lax
from jax.experimental import pallas as pl
from jax.experimental.pallas import tpu as pltpu
from jax.experimental.pallas import tpu_sc as plsc

N = 10000
NPAD = 10240
DIN = 128
DH = 128
DOUT = 64
E = 320000
NCORES = 2
NSUB = 16
NW = NCORES * NSUB
LANES = 128
CHUNK = 128
NCHUNK = 80
NSEG = 2
SEG = NCHUNK // NSEG
EPT = NCHUNK * CHUNK
EPAD = NW * EPT
RPT = NPAD // NSUB

_sc_mesh = plsc.VectorSubcoreMesh(core_axis_name="c", subcore_axis_name="s")


@functools.partial(
    pl.kernel,
    mesh=_sc_mesh,
    out_type=jax.ShapeDtypeStruct((NCORES, NPAD, LANES), jnp.float32),
    scratch_types=[
        pltpu.VMEM((NCHUNK, CHUNK), jnp.int32),
        pltpu.VMEM((CHUNK, LANES), jnp.float32),
        pltpu.VMEM_SHARED((NPAD, LANES), jnp.float32),
    ],
)
def _sc_count(dst_hbm, out_hbm, dst_v, ones_v, cnt_sh):
    c = lax.axis_index("c")
    s = lax.axis_index("s")
    wid = s * NCORES + c

    z16 = jnp.zeros((16,), jnp.float32)

    def _zero(r, _):
        for j in range(LANES // 16):
            ones_v[r, pl.ds(j * 16, 16)] = z16
        return 0

    lax.fori_loop(0, CHUNK, _zero, 0)
    for k in range(RPT // CHUNK):
        pltpu.sync_copy(ones_v, cnt_sh.at[pl.ds(s * RPT + k * CHUNK, CHUNK)])

    o16 = jnp.ones((16,), jnp.float32)

    def _fill(r, _):
        for j in range(LANES // 16):
            ones_v[r, pl.ds(j * 16, 16)] = o16
        return 0

    lax.fori_loop(0, CHUNK, _fill, 0)
    pltpu.sync_copy(dst_hbm.at[wid], dst_v)
    plsc.subcore_barrier()

    def _body(i, _):
        pltpu.sync_copy(ones_v, cnt_sh.at[dst_v.at[i]], add=True)
        return 0

    lax.fori_loop(0, NCHUNK, _body, 0)
    plsc.subcore_barrier()
    pltpu.sync_copy(cnt_sh.at[pl.ds(s * RPT, RPT)],
                    out_hbm.at[c, pl.ds(s * RPT, RPT)])


def _make_agg(D):
    @functools.partial(
        pl.kernel,
        mesh=_sc_mesh,
        out_type=jax.ShapeDtypeStruct((NCORES, NPAD, D), jnp.float32),
        scratch_types=[
            pltpu.VMEM((NCHUNK, CHUNK), jnp.int32),
            pltpu.VMEM((NCHUNK, CHUNK), jnp.int32),
            pltpu.VMEM((CHUNK, D), jnp.float32),
            pltpu.VMEM_SHARED((NPAD, D), jnp.float32),
            pltpu.SemaphoreType.DMA,
        ],
    )
    def _agg(g_hbm, src_hbm, dst_hbm, out_hbm, src_v, dst_v, rows0_v,
             acc_sh, sem0):
        c = lax.axis_index("c")
        s = lax.axis_index("s")
        wid = s * NCORES + c

        z16 = jnp.zeros((16,), jnp.float32)

        def _zero(r, _):
            for j in range(D // 16):
                rows0_v[r, pl.ds(j * 16, 16)] = z16
            return 0

        lax.fori_loop(0, CHUNK, _zero, 0)
        for k in range(RPT // CHUNK):
            pltpu.sync_copy(rows0_v, acc_sh.at[pl.ds(s * RPT + k * CHUNK, CHUNK)])
        plsc.subcore_barrier()

        pltpu.sync_copy(src_hbm.at[wid], src_v)
        pltpu.sync_copy(dst_hbm.at[wid], dst_v)

        def _body(i, _):
            pltpu.async_copy(g_hbm.at[src_v.at[i]], rows0_v, sem0).wait()
            pltpu.sync_copy(rows0_v, acc_sh.at[dst_v.at[i]], add=True)
            return 0

        lax.fori_loop(0, NCHUNK, _body, 0)
        plsc.subcore_barrier()
        pltpu.sync_copy(acc_sh.at[pl.ds(s * RPT, RPT)],
                        out_hbm.at[c, pl.ds(s * RPT, RPT)])

    return _agg


_agg_h = _make_agg(DH)


BLK = 1024
GRID = NPAD // BLK


def _tc_a_body(x_ref, w1_ref, c0_ref, c1_ref, g_ref, dinv_ref):
    deg = 1.0 + c0_ref[:, 0:1] + c1_ref[:, 0:1]
    dinv = lax.rsqrt(deg)
    h = jnp.dot(x_ref[...], w1_ref[...], preferred_element_type=jnp.float32)
    g_ref[...] = h * dinv
    dinv_ref[...] = dinv


_tc_a = pl.pallas_call(
    _tc_a_body,
    grid=(GRID,),
    in_specs=[
        pl.BlockSpec((BLK, DIN), lambda i: (i, 0)),
        pl.BlockSpec((DIN, DH), lambda i: (0, 0)),
        pl.BlockSpec((BLK, LANES), lambda i: (i, 0)),
        pl.BlockSpec((BLK, LANES), lambda i: (i, 0)),
    ],
    out_specs=[
        pl.BlockSpec((BLK, DH), lambda i: (i, 0)),
        pl.BlockSpec((BLK, 1), lambda i: (i, 0)),
    ],
    out_shape=[
        jax.ShapeDtypeStruct((NPAD, DH), jnp.float32),
        jax.ShapeDtypeStruct((NPAD, 1), jnp.float32),
    ],
)


def _tc_b_body(p0_ref, p1_ref, g1_ref, dinv_ref, b1_ref, w2_ref, g2_ref):
    z = (p0_ref[...] + p1_ref[...] + g1_ref[...]) * dinv_ref[...] + b1_ref[...]
    z = jnp.maximum(z, 0.0)
    h2 = jnp.dot(z, w2_ref[...], preferred_element_type=jnp.float32)
    g2_ref[...] = h2 * dinv_ref[...]


_tc_b = pl.pallas_call(
    _tc_b_body,
    grid=(GRID,),
    in_specs=[
        pl.BlockSpec((BLK, DH), lambda i: (i, 0)),
        pl.BlockSpec((BLK, DH), lambda i: (i, 0)),
        pl.BlockSpec((BLK, DH), lambda i: (i, 0)),
        pl.BlockSpec((BLK, 1), lambda i: (i, 0)),
        pl.BlockSpec((1, DH), lambda i: (0, 0)),
        pl.BlockSpec((DH, DH), lambda i: (0, 0)),
    ],
    out_specs=pl.BlockSpec((BLK, DH), lambda i: (i, 0)),
    out_shape=jax.ShapeDtypeStruct((NPAD, DH), jnp.float32),
)


def _tc_c_body(q0_ref, q1_ref, g2_ref, dinv_ref, b2_ref, out_ref):
    zf = q0_ref[...] + q1_ref[...] + g2_ref[...]
    z = zf[:, :DOUT] * dinv_ref[...] + b2_ref[...]
    m = jnp.max(z, axis=1, keepdims=True)
    zs = z - m
    lse = jnp.log(jnp.sum(jnp.exp(zs), axis=1, keepdims=True))
    out_ref[...] = zs - lse


_tc_c = pl.pallas_call(
    _tc_c_body,
    grid=(GRID,),
    in_specs=[
        pl.BlockSpec((BLK, DH), lambda i: (i, 0)),
        pl.BlockSpec((BLK, DH), lambda i: (i, 0)),
        pl.BlockSpec((BLK, DH), lambda i: (i, 0)),
        pl.BlockSpec((BLK, 1), lambda i: (i, 0)),
        pl.BlockSpec((1, DOUT), lambda i: (0, 0)),
    ],
    out_specs=pl.BlockSpec((BLK, DOUT), lambda i: (i, 0)),
    out_shape=jax.ShapeDtypeStruct((NPAD, DOUT), jnp.float32),
)


def kernel(x, edge_index, W1, b1, W2, b2):
    xp = jnp.zeros((NPAD, DIN), jnp.float32).at[:N].set(x)
    src = edge_index[0].astype(jnp.int32)
    dst = edge_index[1].astype(jnp.int32)
    pad = jnp.full((EPAD - E,), N, jnp.int32)
    src3 = jnp.concatenate([src, pad]).reshape(NW, NCHUNK, CHUNK)
    dst3 = jnp.concatenate([dst, pad]).reshape(NW, NCHUNK, CHUNK)

    W2p = jnp.zeros((DH, DH), jnp.float32).at[:, :DOUT].set(W2)

    cnt = _sc_count(dst3)
    g1, dinv = _tc_a(xp, W1, cnt[0], cnt[1])
    p = _agg_h(g1, src3, dst3)
    g2 = _tc_b(p[0], p[1], g1, dinv, b1.reshape(1, DH), W2p)
    q = _agg_h(g2, src3, dst3)
    out = _tc_c(q[0], q[1], g2, dinv, b2.reshape(1, DOUT))
    return out[:N]


# speedup vs baseline: 2.3109x; 2.3047x over previous
import functools

import jax
import jax.numpy as jnp
from jax import lax
from jax.experimental import pallas as pl
from jax.experimental.pallas import tpu as pltpu
from jax.experimental.pallas import tpu_sc as plsc

N = 10000
NPAD = 10240
DIN = 128
DH = 128
DOUT = 64
E = 320000
NCORES = 2
NSUB = 16
NW = NCORES * NSUB
LANES = 128
CHUNK = 128
NCHUNK = 80
NSEG = 2
SEG = NCHUNK // NSEG
EPT = NCHUNK * CHUNK
EPAD = NW * EPT
RPT = NPAD // NSUB

_sc_mesh = plsc.VectorSubcoreMesh(core_axis_name="c", subcore_axis_name="s")


@functools.partial(
    pl.kernel,
    mesh=_sc_mesh,
    out_type=jax.ShapeDtypeStruct((NCORES, NPAD, LANES), jnp.float32),
    scratch_types=[
        pltpu.VMEM((NCHUNK, CHUNK), jnp.int32),
        pltpu.VMEM((CHUNK, LANES), jnp.float32),
        pltpu.VMEM_SHARED((NPAD, LANES), jnp.float32),
    ],
)
def _sc_count(dst_hbm, out_hbm, dst_v, ones_v, cnt_sh):
    c = lax.axis_index("c")
    s = lax.axis_index("s")
    wid = s * NCORES + c

    z16 = jnp.zeros((16,), jnp.float32)

    def _zero(r, _):
        for j in range(LANES // 16):
            ones_v[r, pl.ds(j * 16, 16)] = z16
        return 0

    lax.fori_loop(0, CHUNK, _zero, 0)
    for k in range(RPT // CHUNK):
        pltpu.sync_copy(ones_v, cnt_sh.at[pl.ds(s * RPT + k * CHUNK, CHUNK)])

    o16 = jnp.ones((16,), jnp.float32)

    def _fill(r, _):
        for j in range(LANES // 16):
            ones_v[r, pl.ds(j * 16, 16)] = o16
        return 0

    lax.fori_loop(0, CHUNK, _fill, 0)
    pltpu.sync_copy(dst_hbm.at[wid], dst_v)
    plsc.subcore_barrier()

    def _body(i, _):
        pltpu.sync_copy(ones_v, cnt_sh.at[dst_v.at[i]], add=True)
        return 0

    lax.fori_loop(0, NCHUNK, _body, 0)
    plsc.subcore_barrier()
    pltpu.sync_copy(cnt_sh.at[pl.ds(s * RPT, RPT)],
                    out_hbm.at[c, pl.ds(s * RPT, RPT)])


def _make_agg(D):
    @functools.partial(
        pl.kernel,
        mesh=_sc_mesh,
        out_type=jax.ShapeDtypeStruct((NCORES, NPAD, D), jnp.float32),
        scratch_types=[
            pltpu.VMEM((NCHUNK, CHUNK), jnp.int32),
            pltpu.VMEM((NCHUNK, CHUNK), jnp.int32),
            pltpu.VMEM((CHUNK, D), jnp.float32),
            pltpu.VMEM_SHARED((NPAD, D), jnp.float32),
            pltpu.SemaphoreType.DMA,
        ],
    )
    def _agg(g_hbm, src_hbm, dst_hbm, out_hbm, src_v, dst_v, rows0_v,
             acc_sh, sem0):
        c = lax.axis_index("c")
        s = lax.axis_index("s")
        wid = s * NCORES + c

        z16 = jnp.zeros((16,), jnp.float32)

        def _zero(r, _):
            for j in range(D // 16):
                rows0_v[r, pl.ds(j * 16, 16)] = z16
            return 0

        lax.fori_loop(0, CHUNK, _zero, 0)
        for k in range(RPT // CHUNK):
            pltpu.sync_copy(rows0_v, acc_sh.at[pl.ds(s * RPT + k * CHUNK, CHUNK)])
        plsc.subcore_barrier()

        pltpu.sync_copy(src_hbm.at[wid], src_v)
        pltpu.sync_copy(dst_hbm.at[wid], dst_v)

        def _body(i, _):
            pltpu.async_copy(g_hbm.at[src_v.at[i]], rows0_v, sem0).wait()
            pltpu.sync_copy(rows0_v, acc_sh.at[dst_v.at[i]], add=True)
            return 0

        lax.fori_loop(0, NCHUNK, _body, 0)
        plsc.subcore_barrier()
        pltpu.sync_copy(acc_sh.at[pl.ds(s * RPT, RPT)],
                        out_hbm.at[c, pl.ds(s * RPT, RPT)])

    return _agg


_agg_h = _make_agg(DH)


BLK = 1024
GRID = NPAD // BLK


def _tc_a_body(x_ref, w1_ref, c0_ref, c1_ref, g_ref, dinv_ref):
    deg = 1.0 + c0_ref[:, 0:1] + c1_ref[:, 0:1]
    dinv = lax.rsqrt(deg)
    h = jnp.dot(x_ref[...], w1_ref[...], preferred_element_type=jnp.float32)
    g_ref[...] = h * dinv
    dinv_ref[...] = dinv


_tc_a = pl.pallas_call(
    _tc_a_body,
    grid=(GRID,),
    in_specs=[
        pl.BlockSpec((BLK, DIN), lambda i: (i, 0)),
        pl.BlockSpec((DIN, DH), lambda i: (0, 0)),
        pl.BlockSpec((BLK, LANES), lambda i: (i, 0)),
        pl.BlockSpec((BLK, LANES), lambda i: (i, 0)),
    ],
    out_specs=[
        pl.BlockSpec((BLK, DH), lambda i: (i, 0)),
        pl.BlockSpec((BLK, 1), lambda i: (i, 0)),
    ],
    out_shape=[
        jax.ShapeDtypeStruct((NPAD, DH), jnp.float32),
        jax.ShapeDtypeStruct((NPAD, 1), jnp.float32),
    ],
)


def _tc_b_body(p0_ref, p1_ref, g1_ref, dinv_ref, b1_ref, w2_ref, g2_ref):
    z = (p0_ref[...] + p1_ref[...] + g1_ref[...]) * dinv_ref[...] + b1_ref[...]
    z = jnp.maximum(z, 0.0)
    h2 = jnp.dot(z, w2_ref[...], preferred_element_type=jnp.float32)
    g2_ref[...] = h2 * dinv_ref[...]


_tc_b = pl.pallas_call(
    _tc_b_body,
    grid=(GRID,),
    in_specs=[
        pl.BlockSpec((BLK, DH), lambda i: (i, 0)),
        pl.BlockSpec((BLK, DH), lambda i: (i, 0)),
        pl.BlockSpec((BLK, DH), lambda i: (i, 0)),
        pl.BlockSpec((BLK, 1), lambda i: (i, 0)),
        pl.BlockSpec((1, DH), lambda i: (0, 0)),
        pl.BlockSpec((DH, DH), lambda i: (0, 0)),
    ],
    out_specs=pl.BlockSpec((BLK, DH), lambda i: (i, 0)),
    out_shape=jax.ShapeDtypeStruct((NPAD, DH), jnp.float32),
)


def _tc_c_body(q0_ref, q1_ref, g2_ref, dinv_ref, b2_ref, out_ref):
    zf = q0_ref[...] + q1_ref[...] + g2_ref[...]
    z = zf[:, :DOUT] * dinv_ref[...] + b2_ref[...]
    m = jnp.max(z, axis=1, keepdims=True)
    zs = z - m
    lse = jnp.log(jnp.sum(jnp.exp(zs), axis=1, keepdims=True))
    out_ref[...] = zs - lse


_tc_c = pl.pallas_call(
    _tc_c_body,
    grid=(GRID,),
    in_specs=[
        pl.BlockSpec((BLK, DH), lambda i: (i, 0)),
        pl.BlockSpec((BLK, DH), lambda i: (i, 0)),
        pl.BlockSpec((BLK, DH), lambda i: (i, 0)),
        pl.BlockSpec((BLK, 1), lambda i: (i, 0)),
        pl.BlockSpec((1, DOUT), lambda i: (0, 0)),
    ],
    out_specs=pl.BlockSpec((BLK, DOUT), lambda i: (i, 0)),
    out_shape=jax.ShapeDtypeStruct((NPAD, DOUT), jnp.float32),
)


def kernel(x, edge_index, W1, b1, W2, b2):
    xp = jnp.zeros((NPAD, DIN), jnp.float32).at[:N].set(x)
    src = edge_index[0].astype(jnp.int32)
    dst = edge_index[1].astype(jnp.int32)
    pad = N + jnp.arange(EPAD - E, dtype=jnp.int32) % (NPAD - N)
    src3 = jnp.concatenate([src, pad]).reshape(NW, NCHUNK, CHUNK)
    dst3 = jnp.concatenate([dst, pad]).reshape(NW, NCHUNK, CHUNK)

    W2p = jnp.zeros((DH, DH), jnp.float32).at[:, :DOUT].set(W2)

    cnt = _sc_count(dst3)
    g1, dinv = _tc_a(xp, W1, cnt[0], cnt[1])
    p = _agg_h(g1, src3, dst3)
    g2 = _tc_b(p[0], p[1], g1, dinv, b1.reshape(1, DH), W2p)
    q = _agg_h(g2, src3, dst3)
    out = _tc_c(q[0], q[1], g2, dinv, b2.reshape(1, DOUT))
    return out[:N]


# speedup vs baseline: 3.1209x; 1.3505x over previous
import functools

import jax
import jax.numpy as jnp
from jax import lax
from jax.experimental import pallas as pl
from jax.experimental.pallas import tpu as pltpu
from jax.experimental.pallas import tpu_sc as plsc

N = 10000
NPAD = 10240
DIN = 128
DH = 128
DOUT = 64
E = 320000
NCORES = 2
NSUB = 16
NW = NCORES * NSUB
LANES = 128
CHUNK = 128
NCHUNK = 80
NSEG = 2
SEG = NCHUNK // NSEG
EPT = NCHUNK * CHUNK
EPAD = NW * EPT
RPT = NPAD // NSUB

_sc_mesh = plsc.VectorSubcoreMesh(core_axis_name="c", subcore_axis_name="s")


@functools.partial(
    pl.kernel,
    mesh=_sc_mesh,
    out_type=jax.ShapeDtypeStruct((NCORES, NPAD, LANES), jnp.float32),
    scratch_types=[
        pltpu.VMEM((NCHUNK, CHUNK), jnp.int32),
        pltpu.VMEM((CHUNK, LANES), jnp.float32),
        pltpu.VMEM_SHARED((NPAD, LANES), jnp.float32),
    ],
)
def _sc_count(dst_hbm, out_hbm, dst_v, ones_v, cnt_sh):
    c = lax.axis_index("c")
    s = lax.axis_index("s")
    wid = s * NCORES + c

    z16 = jnp.zeros((16,), jnp.float32)

    def _zero(r, _):
        for j in range(LANES // 16):
            ones_v[r, pl.ds(j * 16, 16)] = z16
        return 0

    lax.fori_loop(0, CHUNK, _zero, 0)
    for k in range(RPT // CHUNK):
        pltpu.sync_copy(ones_v, cnt_sh.at[pl.ds(s * RPT + k * CHUNK, CHUNK)])

    o16 = jnp.ones((16,), jnp.float32)

    def _fill(r, _):
        for j in range(LANES // 16):
            ones_v[r, pl.ds(j * 16, 16)] = o16
        return 0

    lax.fori_loop(0, CHUNK, _fill, 0)
    pltpu.sync_copy(dst_hbm.at[wid], dst_v)
    plsc.subcore_barrier()

    def _body(i, _):
        pltpu.sync_copy(ones_v, cnt_sh.at[dst_v.at[i]], add=True)
        return 0

    lax.fori_loop(0, NCHUNK, _body, 0)
    plsc.subcore_barrier()
    pltpu.sync_copy(cnt_sh.at[pl.ds(s * RPT, RPT)],
                    out_hbm.at[c, pl.ds(s * RPT, RPT)])


def _make_agg(D):
    @functools.partial(
        pl.kernel,
        mesh=_sc_mesh,
        out_type=jax.ShapeDtypeStruct((NCORES, NPAD, D), jnp.float32),
        scratch_types=[
            pltpu.VMEM((SEG, CHUNK), jnp.int32),
            pltpu.VMEM((SEG, CHUNK), jnp.int32),
            pltpu.VMEM((CHUNK, D), jnp.float32),
            pltpu.VMEM((CHUNK, D), jnp.float32),
            pltpu.VMEM_SHARED((NPAD, D), jnp.float32),
            pltpu.SemaphoreType.DMA,
            pltpu.SemaphoreType.DMA,
        ],
    )
    def _agg(g_hbm, src_hbm, dst_hbm, out_hbm, src_v, dst_v, rows0_v, rows1_v,
             acc_sh, sem0, sem1):
        c = lax.axis_index("c")
        s = lax.axis_index("s")
        wid = s * NCORES + c

        z16 = jnp.zeros((16,), jnp.float32)

        def _zero(r, _):
            for j in range(D // 16):
                rows0_v[r, pl.ds(j * 16, 16)] = z16
            return 0

        lax.fori_loop(0, CHUNK, _zero, 0)
        for k in range(RPT // CHUNK):
            pltpu.sync_copy(rows0_v, acc_sh.at[pl.ds(s * RPT + k * CHUNK, CHUNK)])
        plsc.subcore_barrier()

        for seg in range(NSEG):
            pltpu.sync_copy(src_hbm.at[wid, pl.ds(seg * SEG, SEG)], src_v)
            pltpu.sync_copy(dst_hbm.at[wid, pl.ds(seg * SEG, SEG)], dst_v)
            pltpu.async_copy(g_hbm.at[src_v.at[0]], rows0_v, sem0)
            pltpu.async_copy(g_hbm.at[src_v.at[1]], rows1_v, sem1)

            def _body(k, _):
                i0 = 2 * k
                pltpu.make_async_copy(g_hbm.at[src_v.at[i0]], rows0_v,
                                      sem0).wait()
                pltpu.sync_copy(rows0_v, acc_sh.at[dst_v.at[i0]], add=True)

                @pl.when(k < SEG // 2 - 1)
                def _():
                    pltpu.async_copy(g_hbm.at[src_v.at[i0 + 2]], rows0_v, sem0)

                pltpu.make_async_copy(g_hbm.at[src_v.at[i0 + 1]], rows1_v,
                                      sem1).wait()
                pltpu.sync_copy(rows1_v, acc_sh.at[dst_v.at[i0 + 1]], add=True)

                @pl.when(k < SEG // 2 - 1)
                def _():
                    pltpu.async_copy(g_hbm.at[src_v.at[i0 + 3]], rows1_v, sem1)

                return 0

            lax.fori_loop(0, SEG // 2, _body, 0)
        plsc.subcore_barrier()
        pltpu.sync_copy(acc_sh.at[pl.ds(s * RPT, RPT)],
                        out_hbm.at[c, pl.ds(s * RPT, RPT)])

    return _agg


_agg_h = _make_agg(DH)


BLK = 1024
GRID = NPAD // BLK


def _tc_a_body(x_ref, w1_ref, c0_ref, c1_ref, g_ref, dinv_ref):
    deg = 1.0 + c0_ref[:, 0:1] + c1_ref[:, 0:1]
    dinv = lax.rsqrt(deg)
    h = jnp.dot(x_ref[...], w1_ref[...], preferred_element_type=jnp.float32)
    g_ref[...] = h * dinv
    dinv_ref[...] = dinv


_tc_a = pl.pallas_call(
    _tc_a_body,
    grid=(GRID,),
    in_specs=[
        pl.BlockSpec((BLK, DIN), lambda i: (i, 0)),
        pl.BlockSpec((DIN, DH), lambda i: (0, 0)),
        pl.BlockSpec((BLK, LANES), lambda i: (i, 0)),
        pl.BlockSpec((BLK, LANES), lambda i: (i, 0)),
    ],
    out_specs=[
        pl.BlockSpec((BLK, DH), lambda i: (i, 0)),
        pl.BlockSpec((BLK, 1), lambda i: (i, 0)),
    ],
    out_shape=[
        jax.ShapeDtypeStruct((NPAD, DH), jnp.float32),
        jax.ShapeDtypeStruct((NPAD, 1), jnp.float32),
    ],
)


def _tc_b_body(p0_ref, p1_ref, g1_ref, dinv_ref, b1_ref, w2_ref, g2_ref):
    z = (p0_ref[...] + p1_ref[...] + g1_ref[...]) * dinv_ref[...] + b1_ref[...]
    z = jnp.maximum(z, 0.0)
    h2 = jnp.dot(z, w2_ref[...], preferred_element_type=jnp.float32)
    g2_ref[...] = h2 * dinv_ref[...]


_tc_b = pl.pallas_call(
    _tc_b_body,
    grid=(GRID,),
    in_specs=[
        pl.BlockSpec((BLK, DH), lambda i: (i, 0)),
        pl.BlockSpec((BLK, DH), lambda i: (i, 0)),
        pl.BlockSpec((BLK, DH), lambda i: (i, 0)),
        pl.BlockSpec((BLK, 1), lambda i: (i, 0)),
        pl.BlockSpec((1, DH), lambda i: (0, 0)),
        pl.BlockSpec((DH, DH), lambda i: (0, 0)),
    ],
    out_specs=pl.BlockSpec((BLK, DH), lambda i: (i, 0)),
    out_shape=jax.ShapeDtypeStruct((NPAD, DH), jnp.float32),
)


def _tc_c_body(q0_ref, q1_ref, g2_ref, dinv_ref, b2_ref, out_ref):
    zf = q0_ref[...] + q1_ref[...] + g2_ref[...]
    z = zf[:, :DOUT] * dinv_ref[...] + b2_ref[...]
    m = jnp.max(z, axis=1, keepdims=True)
    zs = z - m
    lse = jnp.log(jnp.sum(jnp.exp(zs), axis=1, keepdims=True))
    out_ref[...] = zs - lse


_tc_c = pl.pallas_call(
    _tc_c_body,
    grid=(GRID,),
    in_specs=[
        pl.BlockSpec((BLK, DH), lambda i: (i, 0)),
        pl.BlockSpec((BLK, DH), lambda i: (i, 0)),
        pl.BlockSpec((BLK, DH), lambda i: (i, 0)),
        pl.BlockSpec((BLK, 1), lambda i: (i, 0)),
        pl.BlockSpec((1, DOUT), lambda i: (0, 0)),
    ],
    out_specs=pl.BlockSpec((BLK, DOUT), lambda i: (i, 0)),
    out_shape=jax.ShapeDtypeStruct((NPAD, DOUT), jnp.float32),
)


def kernel(x, edge_index, W1, b1, W2, b2):
    xp = jnp.zeros((NPAD, DIN), jnp.float32).at[:N].set(x)
    src = edge_index[0].astype(jnp.int32)
    dst = edge_index[1].astype(jnp.int32)
    pad = N + jnp.arange(EPAD - E, dtype=jnp.int32) % (NPAD - N)
    src3 = jnp.concatenate([src, pad]).reshape(NW, NCHUNK, CHUNK)
    dst3 = jnp.concatenate([dst, pad]).reshape(NW, NCHUNK, CHUNK)

    W2p = jnp.zeros((DH, DH), jnp.float32).at[:, :DOUT].set(W2)

    cnt = _sc_count(dst3)
    g1, dinv = _tc_a(xp, W1, cnt[0], cnt[1])
    p = _agg_h(g1, src3, dst3)
    g2 = _tc_b(p[0], p[1], g1, dinv, b1.reshape(1, DH), W2p)
    q = _agg_h(g2, src3, dst3)
    out = _tc_c(q[0], q[1], g2, dinv, b2.reshape(1, DOUT))
    return out[:N]


# speedup vs baseline: 3.1213x; 1.0001x over previous
import functools

import jax
import jax.numpy as jnp
from jax import lax
from jax.experimental import pallas as pl
from jax.experimental.pallas import tpu as pltpu
from jax.experimental.pallas import tpu_sc as plsc

N = 10000
NPAD = 10240
DIN = 128
DH = 128
DOUT = 64
E = 320000
NCORES = 2
NSUB = 16
NW = NCORES * NSUB
LANES = 128
CHUNK = 128
NCHUNK = 80
NSEG = 2
SEG = NCHUNK // NSEG
EPT = NCHUNK * CHUNK
EPAD = NW * EPT
RPT = NPAD // NSUB

_sc_mesh = plsc.VectorSubcoreMesh(core_axis_name="c", subcore_axis_name="s")


@functools.partial(
    pl.kernel,
    mesh=_sc_mesh,
    out_type=jax.ShapeDtypeStruct((NCORES, NPAD, LANES), jnp.float32),
    scratch_types=[
        pltpu.VMEM((NCHUNK, CHUNK), jnp.int32),
        pltpu.VMEM((CHUNK, LANES), jnp.float32),
        pltpu.VMEM_SHARED((NPAD, LANES), jnp.float32),
    ],
)
def _sc_count(dst_hbm, out_hbm, dst_v, ones_v, cnt_sh):
    c = lax.axis_index("c")
    s = lax.axis_index("s")
    wid = s * NCORES + c

    z16 = jnp.zeros((16,), jnp.float32)

    def _zero(r, _):
        for j in range(LANES // 16):
            ones_v[r, pl.ds(j * 16, 16)] = z16
        return 0

    lax.fori_loop(0, CHUNK, _zero, 0)
    for k in range(RPT // CHUNK):
        pltpu.sync_copy(ones_v, cnt_sh.at[pl.ds(s * RPT + k * CHUNK, CHUNK)])

    o16 = jnp.ones((16,), jnp.float32)

    def _fill(r, _):
        for j in range(LANES // 16):
            ones_v[r, pl.ds(j * 16, 16)] = o16
        return 0

    lax.fori_loop(0, CHUNK, _fill, 0)
    pltpu.sync_copy(dst_hbm.at[wid], dst_v)
    plsc.subcore_barrier()

    def _body(i, _):
        pltpu.sync_copy(ones_v, cnt_sh.at[dst_v.at[i]], add=True)
        return 0

    lax.fori_loop(0, NCHUNK, _body, 0)
    plsc.subcore_barrier()
    pltpu.sync_copy(cnt_sh.at[pl.ds(s * RPT, RPT)],
                    out_hbm.at[c, pl.ds(s * RPT, RPT)])


def _make_agg(D):
    @functools.partial(
        pl.kernel,
        mesh=_sc_mesh,
        out_type=jax.ShapeDtypeStruct((NCORES, NPAD, D), jnp.float32),
        scratch_types=[
            pltpu.VMEM((SEG, CHUNK), jnp.int32),
            pltpu.VMEM((SEG, CHUNK), jnp.int32),
            pltpu.VMEM((CHUNK, D), jnp.float32),
            pltpu.VMEM((CHUNK, D), jnp.float32),
            pltpu.VMEM_SHARED((NPAD, D), jnp.float32),
            pltpu.SemaphoreType.DMA,
            pltpu.SemaphoreType.DMA,
        ],
    )
    def _agg(g_hbm, src_hbm, dst_hbm, out_hbm, src_v, dst_v, rows0_v, rows1_v,
             acc_sh, sem0, sem1):
        c = lax.axis_index("c")
        s = lax.axis_index("s")
        wid = s * NCORES + c

        z16 = jnp.zeros((16,), jnp.float32)

        def _zero(r, _):
            for j in range(D // 16):
                rows0_v[r, pl.ds(j * 16, 16)] = z16
            return 0

        lax.fori_loop(0, CHUNK, _zero, 0)
        for k in range(RPT // CHUNK):
            pltpu.sync_copy(rows0_v, acc_sh.at[pl.ds(s * RPT + k * CHUNK, CHUNK)])
        plsc.subcore_barrier()

        for seg in range(NSEG):
            pltpu.sync_copy(src_hbm.at[wid, pl.ds(seg * SEG, SEG)], src_v)
            pltpu.sync_copy(dst_hbm.at[wid, pl.ds(seg * SEG, SEG)], dst_v)
            pltpu.async_copy(g_hbm.at[src_v.at[0]], rows0_v, sem0)
            pltpu.async_copy(g_hbm.at[src_v.at[1]], rows1_v, sem1)

            def _body(k, _):
                i0 = 2 * k
                pltpu.make_async_copy(g_hbm.at[src_v.at[i0]], rows0_v,
                                      sem0).wait()
                pltpu.sync_copy(rows0_v, acc_sh.at[dst_v.at[i0]], add=True)

                @pl.when(k < SEG // 2 - 1)
                def _():
                    pltpu.async_copy(g_hbm.at[src_v.at[i0 + 2]], rows0_v, sem0)

                pltpu.make_async_copy(g_hbm.at[src_v.at[i0 + 1]], rows1_v,
                                      sem1).wait()
                pltpu.sync_copy(rows1_v, acc_sh.at[dst_v.at[i0 + 1]], add=True)

                @pl.when(k < SEG // 2 - 1)
                def _():
                    pltpu.async_copy(g_hbm.at[src_v.at[i0 + 3]], rows1_v, sem1)

                return 0

            lax.fori_loop(0, SEG // 2, _body, 0)
        plsc.subcore_barrier()
        pltpu.sync_copy(acc_sh.at[pl.ds(s * RPT, RPT)],
                        out_hbm.at[c, pl.ds(s * RPT, RPT)])

    return _agg


_agg_h = _make_agg(DH)


BLK = 1024
GRID = NPAD // BLK


def _tc_mm_body(x_ref, w1_ref, h_ref):
    h_ref[...] = jnp.dot(x_ref[...], w1_ref[...],
                         preferred_element_type=jnp.float32)


_tc_mm = pl.pallas_call(
    _tc_mm_body,
    grid=(GRID,),
    in_specs=[
        pl.BlockSpec((BLK, DIN), lambda i: (i, 0)),
        pl.BlockSpec((DIN, DH), lambda i: (0, 0)),
    ],
    out_specs=pl.BlockSpec((BLK, DH), lambda i: (i, 0)),
    out_shape=jax.ShapeDtypeStruct((NPAD, DH), jnp.float32),
)


def _tc_a_body(h_ref, c0_ref, c1_ref, g_ref, dinv_ref):
    deg = 1.0 + c0_ref[:, 0:1] + c1_ref[:, 0:1]
    dinv = lax.rsqrt(deg)
    g_ref[...] = h_ref[...] * dinv
    dinv_ref[...] = dinv


_tc_a = pl.pallas_call(
    _tc_a_body,
    grid=(GRID,),
    in_specs=[
        pl.BlockSpec((BLK, DH), lambda i: (i, 0)),
        pl.BlockSpec((BLK, LANES), lambda i: (i, 0)),
        pl.BlockSpec((BLK, LANES), lambda i: (i, 0)),
    ],
    out_specs=[
        pl.BlockSpec((BLK, DH), lambda i: (i, 0)),
        pl.BlockSpec((BLK, 1), lambda i: (i, 0)),
    ],
    out_shape=[
        jax.ShapeDtypeStruct((NPAD, DH), jnp.float32),
        jax.ShapeDtypeStruct((NPAD, 1), jnp.float32),
    ],
)


def _tc_b_body(p0_ref, p1_ref, g1_ref, dinv_ref, b1_ref, w2_ref, g2_ref):
    z = (p0_ref[...] + p1_ref[...] + g1_ref[...]) * dinv_ref[...] + b1_ref[...]
    z = jnp.maximum(z, 0.0)
    h2 = jnp.dot(z, w2_ref[...], preferred_element_type=jnp.float32)
    g2_ref[...] = h2 * dinv_ref[...]


_tc_b = pl.pallas_call(
    _tc_b_body,
    grid=(GRID,),
    in_specs=[
        pl.BlockSpec((BLK, DH), lambda i: (i, 0)),
        pl.BlockSpec((BLK, DH), lambda i: (i, 0)),
        pl.BlockSpec((BLK, DH), lambda i: (i, 0)),
        pl.BlockSpec((BLK, 1), lambda i: (i, 0)),
        pl.BlockSpec((1, DH), lambda i: (0, 0)),
        pl.BlockSpec((DH, DH), lambda i: (0, 0)),
    ],
    out_specs=pl.BlockSpec((BLK, DH), lambda i: (i, 0)),
    out_shape=jax.ShapeDtypeStruct((NPAD, DH), jnp.float32),
)


def _tc_c_body(q0_ref, q1_ref, g2_ref, dinv_ref, b2_ref, out_ref):
    zf = q0_ref[...] + q1_ref[...] + g2_ref[...]
    z = zf[:, :DOUT] * dinv_ref[...] + b2_ref[...]
    m = jnp.max(z, axis=1, keepdims=True)
    zs = z - m
    lse = jnp.log(jnp.sum(jnp.exp(zs), axis=1, keepdims=True))
    out_ref[...] = zs - lse


_tc_c = pl.pallas_call(
    _tc_c_body,
    grid=(GRID,),
    in_specs=[
        pl.BlockSpec((BLK, DH), lambda i: (i, 0)),
        pl.BlockSpec((BLK, DH), lambda i: (i, 0)),
        pl.BlockSpec((BLK, DH), lambda i: (i, 0)),
        pl.BlockSpec((BLK, 1), lambda i: (i, 0)),
        pl.BlockSpec((1, DOUT), lambda i: (0, 0)),
    ],
    out_specs=pl.BlockSpec((BLK, DOUT), lambda i: (i, 0)),
    out_shape=jax.ShapeDtypeStruct((NPAD, DOUT), jnp.float32),
)


def kernel(x, edge_index, W1, b1, W2, b2):
    xp = jnp.zeros((NPAD, DIN), jnp.float32).at[:N].set(x)
    src = edge_index[0].astype(jnp.int32)
    dst = edge_index[1].astype(jnp.int32)
    pad = N + jnp.arange(EPAD - E, dtype=jnp.int32) % (NPAD - N)
    src3 = jnp.concatenate([src, pad]).reshape(NW, NCHUNK, CHUNK)
    dst3 = jnp.concatenate([dst, pad]).reshape(NW, NCHUNK, CHUNK)

    W2p = jnp.zeros((DH, DH), jnp.float32).at[:, :DOUT].set(W2)

    cnt = _sc_count(dst3)
    h1 = _tc_mm(xp, W1)
    g1, dinv = _tc_a(h1, cnt[0], cnt[1])
    p = _agg_h(g1, src3, dst3)
    g2 = _tc_b(p[0], p[1], g1, dinv, b1.reshape(1, DH), W2p)
    q = _agg_h(g2, src3, dst3)
    out = _tc_c(q[0], q[1], g2, dinv, b2.reshape(1, DOUT))
    return out[:N]
